# baseline (speedup 1.0000x reference)
import jax
import jax.numpy as jnp
from jax.experimental import pallas as pl
from jax.experimental.pallas import tpu as pltpu

_BLOCK_N = 1024
_BF16 = jnp.bfloat16


def _shift_rows(a, di):
    if di == 0:
        return a
    pad = jnp.zeros((di, a.shape[1]), a.dtype)
    return jnp.concatenate([a[di:], pad], axis=0)


def _plane_kernel(x_ref, m1_ref, m2_ref, w1r_ref, b1c_ref, b2c_ref,
                  b1f_ref, w2p_ref, b2f_ref, w3p_ref, b3f_ref, o_ref,
                  wa2_ref, wh2_ref, wb2_ref, va2_ref, vb2_ref, w1q_ref,
                  w2b_ref, w3b_ref):
    f32 = jnp.float32

    @pl.when(pl.program_id(0) == 0)
    def _prep():
        wa2_ref[...] = jnp.zeros_like(wa2_ref)
        wh2_ref[...] = jnp.zeros_like(wh2_ref)
        wb2_ref[...] = jnp.zeros_like(wb2_ref)
        va2_ref[...] = jnp.zeros_like(va2_ref)
        vb2_ref[...] = jnp.zeros_like(vb2_ref)
        w1q_ref[...] = jnp.zeros_like(w1q_ref)
        m1 = m1_ref[...].astype(_BF16)
        m2 = m2_ref[...].astype(_BF16)
        w1r = w1r_ref[...].astype(_BF16)
        for k in range(2):
            for e in range(2):
                s = 2 * k + e
                for di in range(5):
                    r = s + di
                    wa2_ref[k, r * 32:(r + 1) * 32,
                            256 * e:256 * (e + 1)] = m1[di]
                s = 4 + 2 * k + e
                for di in range(5):
                    r = s + di
                    if r <= 7:
                        wh2_ref[k, r * 32:(r + 1) * 32,
                                256 * e:256 * (e + 1)] = m1[di]
                    else:
                        wb2_ref[k, (r - 8) * 32:(r - 7) * 32,
                                256 * e:256 * (e + 1)] = m1[di]
                u = 2 * k + e
                for di in range(5):
                    t = u + di
                    if t <= 3:
                        va2_ref[k, t * 128:(t + 1) * 128,
                                256 * e:256 * (e + 1)] = m2[di]
                    else:
                        vb2_ref[k, (t - 4) * 128:(t - 3) * 128,
                                256 * e:256 * (e + 1)] = m2[di]
        for g in range(3):
            for v in range(2):
                if 2 * g + v <= 4:
                    w1q_ref[g, v * 128:(v + 1) * 128, :] = w1r[2 * g + v]
        w2b_ref[...] = w2p_ref[...].astype(_BF16)
        w3b_ref[...] = w3p_ref[...].astype(_BF16)

    xr = x_ref[...].astype(_BF16)
    G = xr.shape[0] * 4
    xw = xr.reshape(G, 256)
    xw1 = _shift_rows(xw, 1)

    b1c = b1c_ref[...]
    b2c = b2c_ref[...]

    def pooled_pair(z, bias):
        lo = jnp.maximum(
            jnp.maximum(z[:, 0:128] + bias[:, 0:128], 0.0),
            jnp.maximum(z[:, 128:256] + bias[:, 128:256], 0.0))
        hi = jnp.maximum(
            jnp.maximum(z[:, 256:384] + bias[:, 0:128], 0.0),
            jnp.maximum(z[:, 384:512] + bias[:, 128:256], 0.0))
        return jnp.maximum(lo, hi).astype(_BF16)

    p1 = [pooled_pair(jnp.dot(xw, wa2_ref[0], preferred_element_type=f32),
                      b1c),
          pooled_pair(jnp.dot(xw, wa2_ref[1], preferred_element_type=f32),
                      b1c),
          pooled_pair(jnp.dot(xw, wh2_ref[0], preferred_element_type=f32)
                      + jnp.dot(xw1, wb2_ref[0], preferred_element_type=f32),
                      b1c),
          pooled_pair(jnp.dot(xw, wh2_ref[1], preferred_element_type=f32)
                      + jnp.dot(xw1, wb2_ref[1], preferred_element_type=f32),
                      b1c)]

    P = jnp.concatenate(p1, axis=1)
    P1 = _shift_rows(P, 1)
    z2a = (jnp.dot(P, va2_ref[0], preferred_element_type=f32)
           + jnp.dot(P1[:, :256], vb2_ref[0, :256, :],
                     preferred_element_type=f32))
    z2b = (jnp.dot(P[:, 256:], va2_ref[1, 256:, :],
                   preferred_element_type=f32)
           + jnp.dot(P1, vb2_ref[1], preferred_element_type=f32))
    p2 = [pooled_pair(z2a, b2c), pooled_pair(z2b, b2c)]

    Q = jnp.concatenate(p2, axis=1)
    F = jnp.dot(Q, w1q_ref[0], preferred_element_type=f32)
    for g in range(1, 3):
        F = F + jnp.dot(_shift_rows(Q, g), w1q_ref[g],
                        preferred_element_type=f32)
    F = jnp.maximum(F + b1f_ref[...], 0.0).astype(_BF16)

    F = F.reshape(G // 4, 512)[:, :128]

    F = jnp.maximum(jnp.dot(F, w2b_ref[...], preferred_element_type=f32)
                    + b2f_ref[...], 0.0).astype(_BF16)
    F = jnp.dot(F, w3b_ref[...], preferred_element_type=f32) + b3f_ref[...]
    o_ref[...] = F[:, :10]


def kernel(m1, b1c, m2, b2c, w1r, b1f, w2p, b2f, w3p, b3f, x_nchw):
    B = x_nchw.shape[0]
    N = _BLOCK_N if B % _BLOCK_N == 0 else B
    G = N * 4
    xf = x_nchw.reshape(B, 1024)

    def full(a):
        if a.ndim == 2:
            return pl.BlockSpec(a.shape, lambda b: (0, 0))
        return pl.BlockSpec(a.shape, lambda b: (0, 0, 0))

    macs_blk = (12 * G * 256 * 256 + 8 * G * 512 * 256 + 3 * G * 256 * 128
                + 2 * G * 128 * 128)
    out = pl.pallas_call(
        _plane_kernel,
        out_shape=jax.ShapeDtypeStruct((B, 10), jnp.float32),
        grid=(B // N,),
        in_specs=[pl.BlockSpec((N, 1024), lambda b: (b, 0)),
                  full(m1), full(m2), full(w1r), full(b1c), full(b2c),
                  full(b1f), full(w2p), full(b2f), full(w3p), full(b3f)],
        out_specs=pl.BlockSpec((N, 10), lambda b: (b, 0)),
        scratch_shapes=[pltpu.VMEM((2, 256, 512), _BF16),
                        pltpu.VMEM((2, 256, 512), _BF16),
                        pltpu.VMEM((2, 256, 512), _BF16),
                        pltpu.VMEM((2, 512, 512), _BF16),
                        pltpu.VMEM((2, 512, 512), _BF16),
                        pltpu.VMEM((3, 256, 128), _BF16),
                        pltpu.VMEM((128, 128), _BF16),
                        pltpu.VMEM((128, 128), _BF16)],
        compiler_params=pltpu.CompilerParams(
            dimension_semantics=("arbitrary",),
            vmem_limit_bytes=64 * 1024 * 1024),
        cost_estimate=pl.CostEstimate(
            flops=2 * macs_blk * (B // N), transcendentals=0,
            bytes_accessed=4 * B * 32 * 32 + 4 * B * 10),
    )(xf, m1, m2, w1r, b1c, b2c, b1f, w2p, b2f, w3p, b3f)
    return out


# speedup vs baseline: 1.0128x; 1.0128x over previous
import jax
import jax.numpy as jnp
from jax.experimental import pallas as pl
from jax.experimental.pallas import tpu as pltpu

_BLOCK_N = 1024
_BF16 = jnp.bfloat16


def _shift_rows(a, di):
    if di == 0:
        return a
    pad = jnp.zeros((di, a.shape[1]), a.dtype)
    return jnp.concatenate([a[di:], pad], axis=0)


def _plane_kernel(x_ref, m1_ref, m2_ref, w1r_ref, b1c_ref, b2c_ref,
                  b1f_ref, w2p_ref, b2f_ref, w3p_ref, b3f_ref, o_ref,
                  wa2_ref, wh2_ref, wb2_ref, va2_ref, vb2_ref, w1q_ref,
                  w2b_ref, w3b_ref):
    f32 = jnp.float32

    @pl.when(pl.program_id(0) == 0)
    def _prep():
        wa2_ref[...] = jnp.zeros_like(wa2_ref)
        wh2_ref[...] = jnp.zeros_like(wh2_ref)
        wb2_ref[...] = jnp.zeros_like(wb2_ref)
        va2_ref[...] = jnp.zeros_like(va2_ref)
        vb2_ref[...] = jnp.zeros_like(vb2_ref)
        w1q_ref[...] = jnp.zeros_like(w1q_ref)
        m1 = m1_ref[...].astype(_BF16)
        m2 = m2_ref[...].astype(_BF16)
        w1r = w1r_ref[...].astype(_BF16)
        for k in range(2):
            for e in range(2):
                s = 2 * k + e
                for di in range(5):
                    r = s + di
                    wa2_ref[k, r * 32:(r + 1) * 32,
                            256 * e:256 * (e + 1)] = m1[di]
                s = 4 + 2 * k + e
                for di in range(5):
                    r = s + di
                    if r <= 7:
                        wh2_ref[k, r * 32:(r + 1) * 32,
                                256 * e:256 * (e + 1)] = m1[di]
                    else:
                        wb2_ref[k, (r - 8) * 32:(r - 7) * 32,
                                256 * e:256 * (e + 1)] = m1[di]
                u = 2 * k + e
                for di in range(5):
                    t = u + di
                    if t <= 3:
                        va2_ref[k, t * 128:(t + 1) * 128,
                                256 * e:256 * (e + 1)] = m2[di]
                    else:
                        vb2_ref[k, (t - 4) * 128:(t - 3) * 128,
                                256 * e:256 * (e + 1)] = m2[di]
        for g in range(3):
            for v in range(2):
                if 2 * g + v <= 4:
                    w1q_ref[g, v * 128:(v + 1) * 128, :] = w1r[2 * g + v]
        w2b_ref[...] = w2p_ref[...].astype(_BF16)
        w3b_ref[...] = w3p_ref[...].astype(_BF16)

    xr = x_ref[...].astype(_BF16)
    G = xr.shape[0] * 4
    xw = xr.reshape(G, 256)
    xw1 = _shift_rows(xw, 1)

    b1c = b1c_ref[...].astype(_BF16)
    b2c = b2c_ref[...].astype(_BF16)
    zero = jnp.array(0.0, _BF16)

    def pooled_pair(z, bias):
        zb = z.astype(_BF16)
        lo = jnp.maximum(
            jnp.maximum(zb[:, 0:128] + bias[:, 0:128], zero),
            jnp.maximum(zb[:, 128:256] + bias[:, 128:256], zero))
        hi = jnp.maximum(
            jnp.maximum(zb[:, 256:384] + bias[:, 0:128], zero),
            jnp.maximum(zb[:, 384:512] + bias[:, 128:256], zero))
        return jnp.maximum(lo, hi)

    p1 = [pooled_pair(jnp.dot(xw, wa2_ref[0], preferred_element_type=f32),
                      b1c),
          pooled_pair(jnp.dot(xw, wa2_ref[1], preferred_element_type=f32),
                      b1c),
          pooled_pair(jnp.dot(xw, wh2_ref[0], preferred_element_type=f32)
                      + jnp.dot(xw1, wb2_ref[0], preferred_element_type=f32),
                      b1c),
          pooled_pair(jnp.dot(xw, wh2_ref[1], preferred_element_type=f32)
                      + jnp.dot(xw1, wb2_ref[1], preferred_element_type=f32),
                      b1c)]

    P = jnp.concatenate(p1, axis=1)
    P1 = _shift_rows(P, 1)
    z2a = (jnp.dot(P, va2_ref[0], preferred_element_type=f32)
           + jnp.dot(P1[:, :256], vb2_ref[0, :256, :],
                     preferred_element_type=f32))
    z2b = (jnp.dot(P[:, 256:], va2_ref[1, 256:, :],
                   preferred_element_type=f32)
           + jnp.dot(P1, vb2_ref[1], preferred_element_type=f32))
    p2 = [pooled_pair(z2a, b2c), pooled_pair(z2b, b2c)]

    Q = jnp.concatenate(p2, axis=1)
    F = jnp.dot(Q, w1q_ref[0], preferred_element_type=f32)
    for g in range(1, 3):
        F = F + jnp.dot(_shift_rows(Q, g), w1q_ref[g],
                        preferred_element_type=f32)
    F = jnp.maximum(F + b1f_ref[...], 0.0).astype(_BF16)

    F = F.reshape(G // 4, 512)[:, :128]

    F = jnp.maximum(jnp.dot(F, w2b_ref[...], preferred_element_type=f32)
                    + b2f_ref[...], 0.0).astype(_BF16)
    F = jnp.dot(F, w3b_ref[...], preferred_element_type=f32) + b3f_ref[...]
    o_ref[...] = F[:, :10]


def kernel(m1, b1c, m2, b2c, w1r, b1f, w2p, b2f, w3p, b3f, x_nchw):
    B = x_nchw.shape[0]
    N = _BLOCK_N if B % _BLOCK_N == 0 else B
    G = N * 4
    xf = x_nchw.reshape(B, 1024)

    def full(a):
        if a.ndim == 2:
            return pl.BlockSpec(a.shape, lambda b: (0, 0))
        return pl.BlockSpec(a.shape, lambda b: (0, 0, 0))

    macs_blk = (12 * G * 256 * 256 + 8 * G * 512 * 256 + 3 * G * 256 * 128
                + 2 * G * 128 * 128)
    out = pl.pallas_call(
        _plane_kernel,
        out_shape=jax.ShapeDtypeStruct((B, 10), jnp.float32),
        grid=(B // N,),
        in_specs=[pl.BlockSpec((N, 1024), lambda b: (b, 0)),
                  full(m1), full(m2), full(w1r), full(b1c), full(b2c),
                  full(b1f), full(w2p), full(b2f), full(w3p), full(b3f)],
        out_specs=pl.BlockSpec((N, 10), lambda b: (b, 0)),
        scratch_shapes=[pltpu.VMEM((2, 256, 512), _BF16),
                        pltpu.VMEM((2, 256, 512), _BF16),
                        pltpu.VMEM((2, 256, 512), _BF16),
                        pltpu.VMEM((2, 512, 512), _BF16),
                        pltpu.VMEM((2, 512, 512), _BF16),
                        pltpu.VMEM((3, 256, 128), _BF16),
                        pltpu.VMEM((128, 128), _BF16),
                        pltpu.VMEM((128, 128), _BF16)],
        compiler_params=pltpu.CompilerParams(
            dimension_semantics=("arbitrary",),
            vmem_limit_bytes=64 * 1024 * 1024),
        cost_estimate=pl.CostEstimate(
            flops=2 * macs_blk * (B // N), transcendentals=0,
            bytes_accessed=4 * B * 32 * 32 + 4 * B * 10),
    )(xf, m1, m2, w1r, b1c, b2c, b1f, w2p, b2f, w3p, b3f)
    return out
